# baseline (speedup 1.0000x reference)
import jax
import jax.numpy as jnp
from jax import lax
from jax.experimental import pallas as pl
from jax.experimental.pallas import tpu as pltpu
from jax.experimental.pallas import tpu_sc as plsc

BATCH = 16384
EMB_DIM = 32
PAD_DIM = 128
NUM_WORKERS = 32
B_PER_W = BATCH // NUM_WORKERS
CB = 64
N_CH = B_PER_W // CB
LANES = 16


def _fire(c, ehp, etp, rfp, rip, h_idx, r_idx, t_idx, bufs, sem):
  b = c % 2
  h1, t1, h2, t2, r1, r2 = bufs
  hi = h_idx.at[c // 2, pl.ds((c % 2) * CB, CB)]
  ri = r_idx.at[c // 2, pl.ds((c % 2) * CB, CB)]
  ti = t_idx.at[c // 2, pl.ds((c % 2) * CB, CB)]
  return [
      pltpu.async_copy(ehp.at[hi], h1.at[b], sem),
      pltpu.async_copy(etp.at[ti], t1.at[b], sem),
      pltpu.async_copy(etp.at[hi], h2.at[b], sem),
      pltpu.async_copy(ehp.at[ti], t2.at[b], sem),
      pltpu.async_copy(rfp.at[ri], r1.at[b], sem),
      pltpu.async_copy(rip.at[ri], r2.at[b], sem),
  ]


def _simple_body(heads_hbm, rels_hbm, tails_hbm, ehp, etp, rfp, rip,
                 out_hbm,
                 h_idx, r_idx, t_idx,
                 h1, t1, h2, t2, r1, r2,
                 out_v, sem):
  wid = lax.axis_index("s") * 2 + lax.axis_index("c")
  base_tile = wid * 4

  pltpu.sync_copy(heads_hbm.at[pl.ds(base_tile, 4)], h_idx)
  pltpu.sync_copy(rels_hbm.at[pl.ds(base_tile, 4)], r_idx)
  pltpu.sync_copy(tails_hbm.at[pl.ds(base_tile, 4)], t_idx)

  bufs = (h1, t1, h2, t2, r1, r2)
  lane = lax.iota(jnp.int32, LANES)
  lo = pl.ds(0, LANES)
  hi_s = pl.ds(LANES, LANES)

  pend = _fire(0, ehp, etp, rfp, rip, h_idx, r_idx, t_idx, bufs, sem)
  for c in range(N_CH):
    nxt = (
        _fire(c + 1, ehp, etp, rfp, rip, h_idx, r_idx, t_idx, bufs, sem)
        if c + 1 < N_CH else []
    )
    for cp in pend:
      cp.wait()
    pend = nxt

    b = c % 2

    def group(i, carry, b=b, c=c):
      acc = jnp.zeros((LANES,), jnp.float32)
      for k in range(LANES):
        row = i * LANES + k
        a0 = (h1[b, row, lo] * r1[b, row, lo] * t1[b, row, lo]
              + h2[b, row, lo] * r2[b, row, lo] * t2[b, row, lo])
        a1 = (h1[b, row, hi_s] * r1[b, row, hi_s] * t1[b, row, hi_s]
              + h2[b, row, hi_s] * r2[b, row, hi_s] * t2[b, row, hi_s])
        acc = jnp.where(lane == k, jnp.sum(a0 + a1), acc)
      out_v[pl.ds(c * CB + i * LANES, LANES)] = acc * 0.5
      return carry

    lax.fori_loop(0, CB // LANES, group, 0)

  pltpu.sync_copy(out_v, out_hbm.at[pl.ds(wid * B_PER_W, B_PER_W)])


@jax.jit
def _simple_sc(heads, rels, tails, eh, et, rf, ri):
  mesh = plsc.VectorSubcoreMesh(core_axis_name="c", subcore_axis_name="s")
  run = pl.kernel(
      _simple_body,
      out_type=jax.ShapeDtypeStruct((BATCH,), jnp.float32),
      mesh=mesh,
      compiler_params=pltpu.CompilerParams(
          needs_layout_passes=False, use_tc_tiling_on_sc=True),
      scratch_types=[
          pltpu.VMEM((4, 128), jnp.int32),
          pltpu.VMEM((4, 128), jnp.int32),
          pltpu.VMEM((4, 128), jnp.int32),
          pltpu.VMEM((2, CB, PAD_DIM), jnp.float32),
          pltpu.VMEM((2, CB, PAD_DIM), jnp.float32),
          pltpu.VMEM((2, CB, PAD_DIM), jnp.float32),
          pltpu.VMEM((2, CB, PAD_DIM), jnp.float32),
          pltpu.VMEM((2, CB, PAD_DIM), jnp.float32),
          pltpu.VMEM((2, CB, PAD_DIM), jnp.float32),
          pltpu.VMEM((B_PER_W,), jnp.float32),
          pltpu.SemaphoreType.DMA,
      ],
  )
  heads2 = heads.astype(jnp.int32).reshape(128, 128)
  rels2 = rels.astype(jnp.int32).reshape(128, 128)
  tails2 = tails.astype(jnp.int32).reshape(128, 128)
  pad = ((0, 0), (0, PAD_DIM - EMB_DIM))
  return run(heads2, rels2, tails2, jnp.pad(eh, pad), jnp.pad(et, pad),
             jnp.pad(rf, pad), jnp.pad(ri, pad))


def kernel(heads, rels, tails, ent_embeds_head, ent_embeds_tail,
           rel_embeds_for, rel_embeds_inv):
  return _simple_sc(heads, rels, tails, ent_embeds_head, ent_embeds_tail,
                    rel_embeds_for, rel_embeds_inv)
